# baseline (speedup 1.0000x reference)
import jax
import jax.numpy as jnp
import numpy as np
from jax import lax
from jax.experimental import pallas as pl
from jax.experimental.pallas import tpu as pltpu
from jax.experimental.pallas import tpu_sc as plsc

_TT = 128
_CH = 128
_NF = 10000
_NB = 2000
_FPAD = 10752
_BPAD = 2560
_RBF = 8
_NW = 32
_BF = _FPAD // _NW
_BB = _BPAD // _NW
_L = 16


def _sc_gather_body(fx_hbm, fy_hbm, ffe_hbm, bx_hbm, by_hbm, bfe_hbm,
                    pf_hbm, pb_hbm,
                    ofx_hbm, ofy_hbm, off_hbm, obx_hbm, oby_hbm, obf_hbm,
                    colf_v, colb_v, idxf_v, idxb_v, outf_v, outb_v):
    wid = lax.axis_index("s") * 2 + lax.axis_index("c")
    basef = wid * _BF
    baseb = wid * _BB
    pltpu.sync_copy(pf_hbm.at[pl.ds(basef, _BF)], idxf_v)
    pltpu.sync_copy(pb_hbm.at[pl.ds(baseb, _BB)], idxb_v)

    def gather_col(col_hbm, out_hbm, col_v, idx_v, out_v, n, base):
        pltpu.sync_copy(col_hbm, col_v)
        for i in range(n // _L):
            vec = idx_v[pl.ds(i * _L, _L)]
            out_v[pl.ds(i * _L, _L)] = plsc.load_gather(col_v, [vec])
        pltpu.sync_copy(out_v, out_hbm.at[pl.ds(base, n)])

    gather_col(fx_hbm, ofx_hbm, colf_v, idxf_v, outf_v, _BF, basef)
    gather_col(fy_hbm, ofy_hbm, colf_v, idxf_v, outf_v, _BF, basef)
    gather_col(ffe_hbm, off_hbm, colf_v, idxf_v, outf_v, _BF, basef)
    gather_col(bx_hbm, obx_hbm, colb_v, idxb_v, outb_v, _BB, baseb)
    gather_col(by_hbm, oby_hbm, colb_v, idxb_v, outb_v, _BB, baseb)
    gather_col(bfe_hbm, obf_hbm, colb_v, idxb_v, outb_v, _BB, baseb)


def _sc_sorted_gather(fx, fy, ffe, bx, by, bfe, pf, pb):
    f32 = jnp.float32
    mesh = plsc.VectorSubcoreMesh(core_axis_name="c", subcore_axis_name="s",
                                  num_cores=2, num_subcores=16)
    return pl.kernel(
        _sc_gather_body,
        out_type=[jax.ShapeDtypeStruct((_FPAD,), f32)] * 3
                 + [jax.ShapeDtypeStruct((_BPAD,), f32)] * 3,
        mesh=mesh,
        scratch_types=[
            pltpu.VMEM((_FPAD,), f32),
            pltpu.VMEM((_BPAD,), f32),
            pltpu.VMEM((_BF,), jnp.int32),
            pltpu.VMEM((_BB,), jnp.int32),
            pltpu.VMEM((_BF,), f32),
            pltpu.VMEM((_BB,), f32),
        ],
        compiler_params=pltpu.CompilerParams(needs_layout_passes=False),
    )(fx, fy, ffe, bx, by, bfe, pf, pb)


_ATAN_C = (0.9999772197188205, -0.3326228337800521, 0.19354039031965328,
           -0.1164264883950182, 0.05264734009558123, -0.011719126877656156)


def _atan2(dy, dx):
    ax = jnp.abs(dx)
    ay = jnp.abs(dy)
    hi = jnp.maximum(ax, ay)
    lo = jnp.minimum(ax, ay)
    a = lo / jnp.maximum(hi, jnp.float32(1e-30))
    s = a * a
    p = jnp.float32(_ATAN_C[5])
    for c in _ATAN_C[4::-1]:
        p = p * s + jnp.float32(c)
    p = p * a
    r = jnp.where(ay > ax, jnp.float32(np.pi / 2) - p, p)
    r = jnp.where(dx < 0.0, jnp.float32(np.pi) - r, r)
    return jnp.where(dy < 0.0, -r, r)


def _pair_acc(acc, tx, ty, sx, sy, sf, wflat, rsq, inv_s):
    dx = sx - tx
    dy = sy - ty
    d2 = dx * dx + dy * dy
    mask = (d2 < rsq).astype(jnp.float32)
    h_inv = jnp.float32((_RBF - 1) / 2.0)
    tu = jnp.minimum(jnp.sqrt(d2) * (2.0 * h_inv * inv_s), jnp.float32(_RBF - 1))
    v = _atan2(dy, dx) * jnp.float32(1.0 / np.pi)
    tv = jnp.clip((v + 1.0) * h_inv, 0.0, jnp.float32(_RBF - 1))
    iu = jnp.minimum(tu.astype(jnp.int32), _RBF - 2)
    iv = jnp.minimum(tv.astype(jnp.int32), _RBF - 2)
    fu = tu - iu.astype(jnp.float32)
    fv = tv - iv.astype(jnp.float32)
    idx = iu * _RBF + iv
    w2d = jnp.broadcast_to(wflat.reshape(1, _RBF * _RBF), (idx.shape[0], _RBF * _RBF))

    def gat(i):
        return jnp.take_along_axis(w2d, i, axis=1, mode="promise_in_bounds")

    g0 = gat(idx)
    g1 = gat(idx + _RBF)
    hi_mask = jnp.int32(-65536)
    w00 = lax.bitcast_convert_type(g0 & hi_mask, jnp.float32)
    w01 = lax.bitcast_convert_type(g0 << 16, jnp.float32)
    w10 = lax.bitcast_convert_type(g1 & hi_mask, jnp.float32)
    w11 = lax.bitcast_convert_type(g1 << 16, jnp.float32)
    t = ((1.0 - fu) * ((1.0 - fv) * w00 + fv * w01)
         + fu * ((1.0 - fv) * w10 + fv * w11))
    return acc + t * (mask * sf)


def _banded_kernel(sup_ref, wf_ref, wb_ref, tx_ref, ty_ref, fcb_ref, bcb_ref,
                   fsx_ref, fsy_ref, fsf_ref, bsx_ref, bsy_ref, bsf_ref,
                   out_ref):
    tx = tx_ref[:, :]
    ty = ty_ref[:, :]
    s = sup_ref[0]
    rsq = s * s
    inv_s = 1.0 / s
    wf = wf_ref[:]
    wb = wb_ref[:]

    lo = jnp.min(tx) - s
    hi = jnp.max(tx) + s

    def chunk_range(cb_row):
        k0 = jnp.maximum(
            jnp.sum((cb_row <= lo).astype(jnp.int32)) - 1, 0)
        k1 = jnp.sum((cb_row < hi).astype(jnp.int32))
        return k0, k1

    fk0, fk1 = chunk_range(fcb_ref[:, :])
    bk0, bk1 = chunk_range(bcb_ref[:, :])

    def fchunk(k, acc):
        sx = fsx_ref[:, pl.ds(k * _CH, _CH)]
        sy = fsy_ref[:, pl.ds(k * _CH, _CH)]
        sf = fsf_ref[:, pl.ds(k * _CH, _CH)]
        return _pair_acc(acc, tx, ty, sx, sy, sf, wf, rsq, inv_s)

    def bchunk(k, acc):
        sx = bsx_ref[:, pl.ds(k * _CH, _CH)]
        sy = bsy_ref[:, pl.ds(k * _CH, _CH)]
        sf = bsf_ref[:, pl.ds(k * _CH, _CH)]
        return _pair_acc(acc, tx, ty, sx, sy, sf, wb, rsq, inv_s)

    def floop3(i, carry):
        a0, a1, a2 = carry
        k = fk0 + 3 * i
        return fchunk(k, a0), fchunk(k + 1, a1), fchunk(k + 2, a2)

    def bloop3(i, carry):
        a0, a1, a2 = carry
        k = bk0 + 3 * i
        return bchunk(k, a0), bchunk(k + 1, a1), bchunk(k + 2, a2)

    z = jnp.zeros((_TT, _CH), jnp.float32)
    a0, a1, a2 = lax.fori_loop(0, (fk1 - fk0 + 2) // 3, floop3, (z, z, z))
    a0, a1, a2 = lax.fori_loop(0, (bk1 - bk0 + 2) // 3, bloop3, (a0, a1, a2))
    out_ref[:, :] = jnp.sum(a0 + a1 + a2, axis=1, keepdims=True)


def kernel(fluidPositions, boundaryPositions, fluidFeatures, boundaryFeatures,
           W_fluid, W_boundary, support):
    f32 = jnp.float32

    def pad_to(x, n, val):
        return jnp.pad(x, (0, n - x.shape[0]), constant_values=val)

    fx_pad = pad_to(fluidPositions[:, 0], _FPAD, 1e9)
    fy_pad = pad_to(fluidPositions[:, 1], _FPAD, 0.0)
    ff_pad = pad_to(fluidFeatures[:, 0], _FPAD, 0.0)
    bx_pad = pad_to(boundaryPositions[:, 0], _BPAD, 1e9)
    by_pad = pad_to(boundaryPositions[:, 1], _BPAD, 0.0)
    bf_pad = pad_to(boundaryFeatures[:, 0], _BPAD, 0.0)
    perm_f = jnp.argsort(fx_pad).astype(jnp.int32)
    perm_b = jnp.argsort(bx_pad).astype(jnp.int32)

    sfx, sfy, sff, sbx, sby, sbf = _sc_sorted_gather(
        fx_pad, fy_pad, ff_pad, bx_pad, by_pad, bf_pad, perm_f, perm_b)

    tx = jnp.concatenate(
        [sfx[:_NF], jnp.full((_FPAD - _NF,), 2.0, f32)]).reshape(_FPAD, 1)
    ty = sfy.reshape(_FPAD, 1)
    fsx = sfx.reshape(1, _FPAD)
    fsy = sfy.reshape(1, _FPAD)
    fsf = sff.reshape(1, _FPAD)
    bsx = sbx.reshape(1, _BPAD)
    bsy = sby.reshape(1, _BPAD)
    bsf = sbf.reshape(1, _BPAD)
    fcb = jnp.pad(sfx[::_CH], (0, 128 - _FPAD // _CH),
                  constant_values=1e30).reshape(1, 128)
    bcb = jnp.pad(sbx[::_CH], (0, 128 - _BPAD // _CH),
                  constant_values=1e30).reshape(1, 128)
    sup = jnp.asarray(support, f32).reshape(1)

    def pack_w(W):
        w = W.reshape(_RBF, _RBF).astype(f32)
        hi = lax.bitcast_convert_type(
            w.astype(jnp.bfloat16), jnp.uint16).astype(jnp.uint32)
        wl = jnp.concatenate([w[:, 1:], w[:, -1:]], axis=1)
        lo = lax.bitcast_convert_type(
            wl.astype(jnp.bfloat16), jnp.uint16).astype(jnp.uint32)
        return lax.bitcast_convert_type(
            (hi << 16) | lo, jnp.int32).reshape(_RBF * _RBF)

    wf = pack_w(W_fluid)
    wb = pack_w(W_boundary)

    grid = (_FPAD // _TT,)
    smem = pl.BlockSpec(memory_space=pltpu.SMEM)
    wspec = pl.BlockSpec((_RBF * _RBF,), lambda i: (0,))
    full_f = pl.BlockSpec((1, _FPAD), lambda i: (0, 0))
    full_b = pl.BlockSpec((1, _BPAD), lambda i: (0, 0))
    cbspec = pl.BlockSpec((1, 128), lambda i: (0, 0))
    tgt = pl.BlockSpec((_TT, 1), lambda i: (i, 0))

    out_sorted = pl.pallas_call(
        _banded_kernel,
        grid=grid,
        in_specs=[smem, wspec, wspec, tgt, tgt, cbspec, cbspec,
                  full_f, full_f, full_f, full_b, full_b, full_b],
        out_specs=pl.BlockSpec((_TT, 1), lambda i: (i, 0)),
        out_shape=jax.ShapeDtypeStruct((_FPAD, 1), f32),
        compiler_params=pltpu.CompilerParams(
            dimension_semantics=("parallel",),
        ),
    )(sup, wf, wb, tx, ty, fcb, bcb, fsx, fsy, fsf, bsx, bsy, bsf)

    return jnp.zeros((_NF, 1), f32).at[perm_f[:_NF]].set(out_sorted[:_NF])


# speedup vs baseline: 1.0232x; 1.0232x over previous
import jax
import jax.numpy as jnp
import numpy as np
from jax import lax
from jax.experimental import pallas as pl
from jax.experimental.pallas import tpu as pltpu
from jax.experimental.pallas import tpu_sc as plsc

_TT = 128
_CH = 128
_NF = 10000
_NB = 2000
_FPAD = 10752
_BPAD = 2560
_RBF = 8
_NW = 32
_BF = _FPAD // _NW
_BB = _BPAD // _NW
_L = 16


def _sc_gather_body(fx_hbm, fy_hbm, ffe_hbm, bx_hbm, by_hbm, bfe_hbm,
                    pf_hbm, pb_hbm,
                    ofx_hbm, ofy_hbm, off_hbm, obx_hbm, oby_hbm, obf_hbm,
                    colf_v, colb_v, idxf_v, idxb_v, outf_v, outb_v):
    wid = lax.axis_index("s") * 2 + lax.axis_index("c")
    basef = wid * _BF
    baseb = wid * _BB
    pltpu.sync_copy(pf_hbm.at[pl.ds(basef, _BF)], idxf_v)
    pltpu.sync_copy(pb_hbm.at[pl.ds(baseb, _BB)], idxb_v)

    def gather_col(col_hbm, out_hbm, col_v, idx_v, out_v, n, base):
        pltpu.sync_copy(col_hbm, col_v)
        for i in range(n // _L):
            vec = idx_v[pl.ds(i * _L, _L)]
            out_v[pl.ds(i * _L, _L)] = plsc.load_gather(col_v, [vec])
        pltpu.sync_copy(out_v, out_hbm.at[pl.ds(base, n)])

    gather_col(fx_hbm, ofx_hbm, colf_v, idxf_v, outf_v, _BF, basef)
    gather_col(fy_hbm, ofy_hbm, colf_v, idxf_v, outf_v, _BF, basef)
    gather_col(ffe_hbm, off_hbm, colf_v, idxf_v, outf_v, _BF, basef)
    gather_col(bx_hbm, obx_hbm, colb_v, idxb_v, outb_v, _BB, baseb)
    gather_col(by_hbm, oby_hbm, colb_v, idxb_v, outb_v, _BB, baseb)
    gather_col(bfe_hbm, obf_hbm, colb_v, idxb_v, outb_v, _BB, baseb)


def _sc_sorted_gather(fx, fy, ffe, bx, by, bfe, pf, pb):
    f32 = jnp.float32
    mesh = plsc.VectorSubcoreMesh(core_axis_name="c", subcore_axis_name="s",
                                  num_cores=2, num_subcores=16)
    return pl.kernel(
        _sc_gather_body,
        out_type=[jax.ShapeDtypeStruct((_FPAD,), f32)] * 3
                 + [jax.ShapeDtypeStruct((_BPAD,), f32)] * 3,
        mesh=mesh,
        scratch_types=[
            pltpu.VMEM((_FPAD,), f32),
            pltpu.VMEM((_BPAD,), f32),
            pltpu.VMEM((_BF,), jnp.int32),
            pltpu.VMEM((_BB,), jnp.int32),
            pltpu.VMEM((_BF,), f32),
            pltpu.VMEM((_BB,), f32),
        ],
        compiler_params=pltpu.CompilerParams(needs_layout_passes=False),
    )(fx, fy, ffe, bx, by, bfe, pf, pb)


_ATAN_C = (0.9999772197188205, -0.3326228337800521, 0.19354039031965328,
           -0.1164264883950182, 0.05264734009558123, -0.011719126877656156)


def _atan2(dy, dx):
    ax = jnp.abs(dx)
    ay = jnp.abs(dy)
    hi = jnp.maximum(ax, ay)
    lo = jnp.minimum(ax, ay)
    a = lo / jnp.maximum(hi, jnp.float32(1e-30))
    s = a * a
    p = jnp.float32(_ATAN_C[5])
    for c in _ATAN_C[4::-1]:
        p = p * s + jnp.float32(c)
    p = p * a
    r = jnp.where(ay > ax, jnp.float32(np.pi / 2) - p, p)
    r = jnp.where(dx < 0.0, jnp.float32(np.pi) - r, r)
    return jnp.where(dy < 0.0, -r, r)


def _pair_acc(acc, tx, ty, sx, sy, sf, wflat, rsq, inv_s):
    dx = sx - tx
    dy = sy - ty
    d2 = dx * dx + dy * dy
    mask = (d2 < rsq).astype(jnp.float32)
    h_inv = jnp.float32((_RBF - 1) / 2.0)
    tu = jnp.minimum(jnp.sqrt(d2) * (2.0 * h_inv * inv_s), jnp.float32(_RBF - 1))
    v = _atan2(dy, dx) * jnp.float32(1.0 / np.pi)
    tv = jnp.clip((v + 1.0) * h_inv, 0.0, jnp.float32(_RBF - 1))
    iu = jnp.minimum(tu.astype(jnp.int32), _RBF - 2)
    iv = jnp.minimum(tv.astype(jnp.int32), _RBF - 2)
    fu = tu - iu.astype(jnp.float32)
    fv = tv - iv.astype(jnp.float32)
    idx = iu * _RBF + iv
    w2d = jnp.broadcast_to(wflat.reshape(1, _RBF * _RBF), (idx.shape[0], _RBF * _RBF))

    def gat(i):
        return jnp.take_along_axis(w2d, i, axis=1, mode="promise_in_bounds")

    g0 = gat(idx)
    g1 = gat(idx + _RBF)
    hi_mask = jnp.int32(-65536)
    w00 = lax.bitcast_convert_type(g0 & hi_mask, jnp.float32)
    w01 = lax.bitcast_convert_type(g0 << 16, jnp.float32)
    w10 = lax.bitcast_convert_type(g1 & hi_mask, jnp.float32)
    w11 = lax.bitcast_convert_type(g1 << 16, jnp.float32)
    t = ((1.0 - fu) * ((1.0 - fv) * w00 + fv * w01)
         + fu * ((1.0 - fv) * w10 + fv * w11))
    return acc + t * (mask * sf)


def _banded_kernel(sup_ref, wf_ref, wb_ref, tx_ref, ty_ref, fcb_ref, bcb_ref,
                   fsx_ref, fsy_ref, fsf_ref, bsx_ref, bsy_ref, bsf_ref,
                   out_ref):
    tx = tx_ref[:, :]
    ty = ty_ref[:, :]
    s = sup_ref[0]
    rsq = s * s
    inv_s = 1.0 / s
    wf = wf_ref[:]
    wb = wb_ref[:]

    lo = jnp.min(tx) - s
    hi = jnp.max(tx) + s

    def chunk_range(cb_row):
        k0 = jnp.maximum(
            jnp.sum((cb_row <= lo).astype(jnp.int32)) - 1, 0)
        k1 = jnp.sum((cb_row < hi).astype(jnp.int32))
        return k0, k1

    fk0, fk1 = chunk_range(fcb_ref[:, :])
    bk0, bk1 = chunk_range(bcb_ref[:, :])

    def fchunk(k, acc):
        sx = fsx_ref[:, pl.ds(k * _CH, _CH)]
        sy = fsy_ref[:, pl.ds(k * _CH, _CH)]
        sf = fsf_ref[:, pl.ds(k * _CH, _CH)]
        return _pair_acc(acc, tx, ty, sx, sy, sf, wf, rsq, inv_s)

    def bchunk(k, acc):
        sx = bsx_ref[:, pl.ds(k * _CH, _CH)]
        sy = bsy_ref[:, pl.ds(k * _CH, _CH)]
        sf = bsf_ref[:, pl.ds(k * _CH, _CH)]
        return _pair_acc(acc, tx, ty, sx, sy, sf, wb, rsq, inv_s)

    def floop2(i, carry):
        a0, a1 = carry
        k = fk0 + 2 * i
        return fchunk(k, a0), fchunk(k + 1, a1)

    def bloop2(i, carry):
        a0, a1 = carry
        k = bk0 + 2 * i
        return bchunk(k, a0), bchunk(k + 1, a1)

    z = jnp.zeros((_TT, _CH), jnp.float32)
    a0, a1 = lax.fori_loop(0, (fk1 - fk0 + 1) // 2, floop2, (z, z))
    a0, a1 = lax.fori_loop(0, (bk1 - bk0 + 1) // 2, bloop2, (a0, a1))
    out_ref[:, :] = jnp.sum(a0 + a1, axis=1, keepdims=True)


def kernel(fluidPositions, boundaryPositions, fluidFeatures, boundaryFeatures,
           W_fluid, W_boundary, support):
    f32 = jnp.float32

    def pad_to(x, n, val):
        return jnp.pad(x, (0, n - x.shape[0]), constant_values=val)

    fx_pad = pad_to(fluidPositions[:, 0], _FPAD, 1e9)
    fy_pad = pad_to(fluidPositions[:, 1], _FPAD, 0.0)
    ff_pad = pad_to(fluidFeatures[:, 0], _FPAD, 0.0)
    bx_pad = pad_to(boundaryPositions[:, 0], _BPAD, 1e9)
    by_pad = pad_to(boundaryPositions[:, 1], _BPAD, 0.0)
    bf_pad = pad_to(boundaryFeatures[:, 0], _BPAD, 0.0)
    perm_f = jnp.argsort(fx_pad).astype(jnp.int32)
    perm_b = jnp.argsort(bx_pad).astype(jnp.int32)

    sfx, sfy, sff, sbx, sby, sbf = _sc_sorted_gather(
        fx_pad, fy_pad, ff_pad, bx_pad, by_pad, bf_pad, perm_f, perm_b)

    tx = jnp.concatenate(
        [sfx[:_NF], jnp.full((_FPAD - _NF,), 2.0, f32)]).reshape(_FPAD, 1)
    ty = sfy.reshape(_FPAD, 1)
    fsx = sfx.reshape(1, _FPAD)
    fsy = sfy.reshape(1, _FPAD)
    fsf = sff.reshape(1, _FPAD)
    bsx = sbx.reshape(1, _BPAD)
    bsy = sby.reshape(1, _BPAD)
    bsf = sbf.reshape(1, _BPAD)
    fcb = jnp.pad(sfx[::_CH], (0, 128 - _FPAD // _CH),
                  constant_values=1e30).reshape(1, 128)
    bcb = jnp.pad(sbx[::_CH], (0, 128 - _BPAD // _CH),
                  constant_values=1e30).reshape(1, 128)
    sup = jnp.asarray(support, f32).reshape(1)

    def pack_w(W):
        w = W.reshape(_RBF, _RBF).astype(f32)
        hi = lax.bitcast_convert_type(
            w.astype(jnp.bfloat16), jnp.uint16).astype(jnp.uint32)
        wl = jnp.concatenate([w[:, 1:], w[:, -1:]], axis=1)
        lo = lax.bitcast_convert_type(
            wl.astype(jnp.bfloat16), jnp.uint16).astype(jnp.uint32)
        return lax.bitcast_convert_type(
            (hi << 16) | lo, jnp.int32).reshape(_RBF * _RBF)

    wf = pack_w(W_fluid)
    wb = pack_w(W_boundary)

    grid = (_FPAD // _TT,)
    smem = pl.BlockSpec(memory_space=pltpu.SMEM)
    wspec = pl.BlockSpec((_RBF * _RBF,), lambda i: (0,))
    full_f = pl.BlockSpec((1, _FPAD), lambda i: (0, 0))
    full_b = pl.BlockSpec((1, _BPAD), lambda i: (0, 0))
    cbspec = pl.BlockSpec((1, 128), lambda i: (0, 0))
    tgt = pl.BlockSpec((_TT, 1), lambda i: (i, 0))

    out_sorted = pl.pallas_call(
        _banded_kernel,
        grid=grid,
        in_specs=[smem, wspec, wspec, tgt, tgt, cbspec, cbspec,
                  full_f, full_f, full_f, full_b, full_b, full_b],
        out_specs=pl.BlockSpec((_TT, 1), lambda i: (i, 0)),
        out_shape=jax.ShapeDtypeStruct((_FPAD, 1), f32),
        compiler_params=pltpu.CompilerParams(
            dimension_semantics=("parallel",),
        ),
    )(sup, wf, wb, tx, ty, fcb, bcb, fsx, fsy, fsf, bsx, bsy, bsf)

    return jnp.zeros((_NF, 1), f32).at[perm_f[:_NF]].set(out_sorted[:_NF])


# speedup vs baseline: 1.1184x; 1.0931x over previous
import jax
import jax.numpy as jnp
import numpy as np
from jax import lax
from jax.experimental import pallas as pl
from jax.experimental.pallas import tpu as pltpu
from jax.experimental.pallas import tpu_sc as plsc

_TT = 256
_CH = 128
_NF = 10000
_NB = 2000
_FPAD = 10752
_BPAD = 2560
_RBF = 8
_NW = 32
_BF = _FPAD // _NW
_BB = _BPAD // _NW
_L = 16


def _sc_gather_body(fx_hbm, fy_hbm, ffe_hbm, bx_hbm, by_hbm, bfe_hbm,
                    pf_hbm, pb_hbm,
                    ofx_hbm, ofy_hbm, off_hbm, obx_hbm, oby_hbm, obf_hbm,
                    colf_v, colb_v, idxf_v, idxb_v, outf_v, outb_v):
    wid = lax.axis_index("s") * 2 + lax.axis_index("c")
    basef = wid * _BF
    baseb = wid * _BB
    pltpu.sync_copy(pf_hbm.at[pl.ds(basef, _BF)], idxf_v)
    pltpu.sync_copy(pb_hbm.at[pl.ds(baseb, _BB)], idxb_v)

    def gather_col(col_hbm, out_hbm, col_v, idx_v, out_v, n, base):
        pltpu.sync_copy(col_hbm, col_v)
        for i in range(n // _L):
            vec = idx_v[pl.ds(i * _L, _L)]
            out_v[pl.ds(i * _L, _L)] = plsc.load_gather(col_v, [vec])
        pltpu.sync_copy(out_v, out_hbm.at[pl.ds(base, n)])

    gather_col(fx_hbm, ofx_hbm, colf_v, idxf_v, outf_v, _BF, basef)
    gather_col(fy_hbm, ofy_hbm, colf_v, idxf_v, outf_v, _BF, basef)
    gather_col(ffe_hbm, off_hbm, colf_v, idxf_v, outf_v, _BF, basef)
    gather_col(bx_hbm, obx_hbm, colb_v, idxb_v, outb_v, _BB, baseb)
    gather_col(by_hbm, oby_hbm, colb_v, idxb_v, outb_v, _BB, baseb)
    gather_col(bfe_hbm, obf_hbm, colb_v, idxb_v, outb_v, _BB, baseb)


def _sc_sorted_gather(fx, fy, ffe, bx, by, bfe, pf, pb):
    f32 = jnp.float32
    mesh = plsc.VectorSubcoreMesh(core_axis_name="c", subcore_axis_name="s",
                                  num_cores=2, num_subcores=16)
    return pl.kernel(
        _sc_gather_body,
        out_type=[jax.ShapeDtypeStruct((_FPAD,), f32)] * 3
                 + [jax.ShapeDtypeStruct((_BPAD,), f32)] * 3,
        mesh=mesh,
        scratch_types=[
            pltpu.VMEM((_FPAD,), f32),
            pltpu.VMEM((_BPAD,), f32),
            pltpu.VMEM((_BF,), jnp.int32),
            pltpu.VMEM((_BB,), jnp.int32),
            pltpu.VMEM((_BF,), f32),
            pltpu.VMEM((_BB,), f32),
        ],
        compiler_params=pltpu.CompilerParams(needs_layout_passes=False),
    )(fx, fy, ffe, bx, by, bfe, pf, pb)


_ATAN_C = (0.9999772197188205, -0.3326228337800521, 0.19354039031965328,
           -0.1164264883950182, 0.05264734009558123, -0.011719126877656156)


def _atan2(dy, dx):
    ax = jnp.abs(dx)
    ay = jnp.abs(dy)
    hi = jnp.maximum(ax, ay)
    lo = jnp.minimum(ax, ay)
    a = lo / jnp.maximum(hi, jnp.float32(1e-30))
    s = a * a
    p = jnp.float32(_ATAN_C[5])
    for c in _ATAN_C[4::-1]:
        p = p * s + jnp.float32(c)
    p = p * a
    r = jnp.where(ay > ax, jnp.float32(np.pi / 2) - p, p)
    r = jnp.where(dx < 0.0, jnp.float32(np.pi) - r, r)
    return jnp.where(dy < 0.0, -r, r)


def _pair_acc(acc, tx, ty, sx, sy, sf, wflat, rsq, inv_s):
    dx = sx - tx
    dy = sy - ty
    d2 = dx * dx + dy * dy
    mask = (d2 < rsq).astype(jnp.float32)
    h_inv = jnp.float32((_RBF - 1) / 2.0)
    tu = jnp.minimum(jnp.sqrt(d2) * (2.0 * h_inv * inv_s), jnp.float32(_RBF - 1))
    v = _atan2(dy, dx) * jnp.float32(1.0 / np.pi)
    tv = jnp.clip((v + 1.0) * h_inv, 0.0, jnp.float32(_RBF - 1))
    iu = jnp.minimum(tu.astype(jnp.int32), _RBF - 2)
    iv = jnp.minimum(tv.astype(jnp.int32), _RBF - 2)
    fu = tu - iu.astype(jnp.float32)
    fv = tv - iv.astype(jnp.float32)
    idx = iu * _RBF + iv
    w2d = jnp.broadcast_to(wflat.reshape(1, _RBF * _RBF), (idx.shape[0], _RBF * _RBF))

    def gat(i):
        return jnp.take_along_axis(w2d, i, axis=1, mode="promise_in_bounds")

    g0 = gat(idx)
    g1 = gat(idx + _RBF)
    hi_mask = jnp.int32(-65536)
    w00 = lax.bitcast_convert_type(g0 & hi_mask, jnp.float32)
    w01 = lax.bitcast_convert_type(g0 << 16, jnp.float32)
    w10 = lax.bitcast_convert_type(g1 & hi_mask, jnp.float32)
    w11 = lax.bitcast_convert_type(g1 << 16, jnp.float32)
    t = ((1.0 - fu) * ((1.0 - fv) * w00 + fv * w01)
         + fu * ((1.0 - fv) * w10 + fv * w11))
    return acc + t * (mask * sf)


def _banded_kernel(sup_ref, wf_ref, wb_ref, tx_ref, ty_ref, fcb_ref, bcb_ref,
                   fsx_ref, fsy_ref, fsf_ref, bsx_ref, bsy_ref, bsf_ref,
                   out_ref):
    tx = tx_ref[:, :]
    ty = ty_ref[:, :]
    s = sup_ref[0]
    rsq = s * s
    inv_s = 1.0 / s
    wf = wf_ref[:]
    wb = wb_ref[:]

    lo = jnp.min(tx) - s
    hi = jnp.max(tx) + s

    def chunk_range(cb_row):
        k0 = jnp.maximum(
            jnp.sum((cb_row <= lo).astype(jnp.int32)) - 1, 0)
        k1 = jnp.sum((cb_row < hi).astype(jnp.int32))
        return k0, k1

    fk0, fk1 = chunk_range(fcb_ref[:, :])
    bk0, bk1 = chunk_range(bcb_ref[:, :])

    def fchunk(k, acc):
        sx = fsx_ref[:, pl.ds(k * _CH, _CH)]
        sy = fsy_ref[:, pl.ds(k * _CH, _CH)]
        sf = fsf_ref[:, pl.ds(k * _CH, _CH)]
        return _pair_acc(acc, tx, ty, sx, sy, sf, wf, rsq, inv_s)

    def bchunk(k, acc):
        sx = bsx_ref[:, pl.ds(k * _CH, _CH)]
        sy = bsy_ref[:, pl.ds(k * _CH, _CH)]
        sf = bsf_ref[:, pl.ds(k * _CH, _CH)]
        return _pair_acc(acc, tx, ty, sx, sy, sf, wb, rsq, inv_s)

    def floop2(i, carry):
        a0, a1 = carry
        k = fk0 + 2 * i
        return fchunk(k, a0), fchunk(k + 1, a1)

    def bloop2(i, carry):
        a0, a1 = carry
        k = bk0 + 2 * i
        return bchunk(k, a0), bchunk(k + 1, a1)

    z = jnp.zeros((_TT, _CH), jnp.float32)
    a0, a1 = lax.fori_loop(0, (fk1 - fk0 + 1) // 2, floop2, (z, z))
    a0, a1 = lax.fori_loop(0, (bk1 - bk0 + 1) // 2, bloop2, (a0, a1))
    out_ref[:, :] = jnp.sum(a0 + a1, axis=1, keepdims=True)


def kernel(fluidPositions, boundaryPositions, fluidFeatures, boundaryFeatures,
           W_fluid, W_boundary, support):
    f32 = jnp.float32

    def pad_to(x, n, val):
        return jnp.pad(x, (0, n - x.shape[0]), constant_values=val)

    fx_pad = pad_to(fluidPositions[:, 0], _FPAD, 1e9)
    fy_pad = pad_to(fluidPositions[:, 1], _FPAD, 0.0)
    ff_pad = pad_to(fluidFeatures[:, 0], _FPAD, 0.0)
    bx_pad = pad_to(boundaryPositions[:, 0], _BPAD, 1e9)
    by_pad = pad_to(boundaryPositions[:, 1], _BPAD, 0.0)
    bf_pad = pad_to(boundaryFeatures[:, 0], _BPAD, 0.0)
    perm_f = jnp.argsort(fx_pad).astype(jnp.int32)
    perm_b = jnp.argsort(bx_pad).astype(jnp.int32)

    sfx, sfy, sff, sbx, sby, sbf = _sc_sorted_gather(
        fx_pad, fy_pad, ff_pad, bx_pad, by_pad, bf_pad, perm_f, perm_b)

    tx = jnp.concatenate(
        [sfx[:_NF], jnp.full((_FPAD - _NF,), 2.0, f32)]).reshape(_FPAD, 1)
    ty = sfy.reshape(_FPAD, 1)
    fsx = sfx.reshape(1, _FPAD)
    fsy = sfy.reshape(1, _FPAD)
    fsf = sff.reshape(1, _FPAD)
    bsx = sbx.reshape(1, _BPAD)
    bsy = sby.reshape(1, _BPAD)
    bsf = sbf.reshape(1, _BPAD)
    fcb = jnp.pad(sfx[::_CH], (0, 128 - _FPAD // _CH),
                  constant_values=1e30).reshape(1, 128)
    bcb = jnp.pad(sbx[::_CH], (0, 128 - _BPAD // _CH),
                  constant_values=1e30).reshape(1, 128)
    sup = jnp.asarray(support, f32).reshape(1)

    def pack_w(W):
        w = W.reshape(_RBF, _RBF).astype(f32)
        hi = lax.bitcast_convert_type(
            w.astype(jnp.bfloat16), jnp.uint16).astype(jnp.uint32)
        wl = jnp.concatenate([w[:, 1:], w[:, -1:]], axis=1)
        lo = lax.bitcast_convert_type(
            wl.astype(jnp.bfloat16), jnp.uint16).astype(jnp.uint32)
        return lax.bitcast_convert_type(
            (hi << 16) | lo, jnp.int32).reshape(_RBF * _RBF)

    wf = pack_w(W_fluid)
    wb = pack_w(W_boundary)

    grid = (_FPAD // _TT,)
    smem = pl.BlockSpec(memory_space=pltpu.SMEM)
    wspec = pl.BlockSpec((_RBF * _RBF,), lambda i: (0,))
    full_f = pl.BlockSpec((1, _FPAD), lambda i: (0, 0))
    full_b = pl.BlockSpec((1, _BPAD), lambda i: (0, 0))
    cbspec = pl.BlockSpec((1, 128), lambda i: (0, 0))
    tgt = pl.BlockSpec((_TT, 1), lambda i: (i, 0))

    out_sorted = pl.pallas_call(
        _banded_kernel,
        grid=grid,
        in_specs=[smem, wspec, wspec, tgt, tgt, cbspec, cbspec,
                  full_f, full_f, full_f, full_b, full_b, full_b],
        out_specs=pl.BlockSpec((_TT, 1), lambda i: (i, 0)),
        out_shape=jax.ShapeDtypeStruct((_FPAD, 1), f32),
        compiler_params=pltpu.CompilerParams(
            dimension_semantics=("parallel",),
        ),
    )(sup, wf, wb, tx, ty, fcb, bcb, fsx, fsy, fsf, bsx, bsy, bsf)

    return jnp.zeros((_NF, 1), f32).at[perm_f[:_NF]].set(out_sorted[:_NF])
